# baseline (speedup 1.0000x reference)
import jax
import jax.numpy as jnp
from jax import lax
from jax.experimental import pallas as pl
from jax.experimental.pallas import tpu as pltpu
from jax.experimental.pallas import tpu_sc as plsc

NUM_USER = 6000
NUM_ITEM = 4000
N_NODES = NUM_USER + NUM_ITEM
LATENT_DIM = 256
HALF = LATENT_DIM // 2
N_EDGES = 160000
TILES = 16
EDGES_PER_TILE = N_EDGES // TILES
CHUNK = 80
N_REAL = EDGES_PER_TILE // CHUNK
N_CHUNKS = 128
GRP = 24
N_GRP = 5
TAILG = N_CHUNKS - N_GRP * GRP
NBUF = 4
RCH = 40
N_RCH = N_NODES // RCH
RPASS = (N_RCH + TILES - 1) // TILES
N_LAYERS = 3


def _body(x0, col4, row3, w3, out, x1b, x2b,
          acc, m0, m1, m2, m3, colb, rowb, wb,
          g0s, g1s, g2s, g3s, s0s, s1s, s2s, s3s):
    c = lax.axis_index("c")
    s = lax.axis_index("s")
    base = c * N_NODES

    m = (m0, m1, m2, m3)
    gs = (g0s, g1s, g2s, g3s)
    ss = (s0s, s1s, s2s, s3s)

    zero16 = jnp.zeros((16,), jnp.float32)

    def zfill(i, _):
        for g in range(HALF // 16):
            m3[i, pl.ds(g * 16, 16)] = zero16
        return _

    def my_chunks(n_ch, npass, step, fn):
        for k in range(npass):
            ch = k * TILES + s
            @pl.when(ch < n_ch)
            def _():
                fn(pl.multiple_of(ch * step, step))

    lax.fori_loop(0, RCH, zfill, None)
    my_chunks(N_RCH, RPASS, RCH,
              lambda r0: pltpu.async_copy(m3.at[pl.ds(0, RCH)],
                                          acc.at[pl.ds(r0, RCH)], s0s))
    my_chunks(N_RCH, RPASS, RCH,
              lambda r0: pltpu.make_async_copy(m3.at[pl.ds(0, RCH)],
                                               acc.at[pl.ds(r0, RCH)], s0s).wait())
    plsc.subcore_barrier()

    def scale_chunk(buf, j):
        def scale16(q, _):
            w16 = wb[j, pl.ds(q * 16, 16)]
            for t in range(16):
                wbv = jnp.full((16,), w16[t], jnp.float32)
                e = q * 16 + t
                for g in range(HALF // 16):
                    sl = pl.ds(g * 16, 16)
                    buf[e, sl] = buf[e, sl] * wbv
            return _

        lax.fori_loop(0, CHUNK // 16, scale16, None)

    for layer in range(N_LAYERS):
        src = (x0, x1b, x2b)[layer]

        def load_group(gg0, glen):
            d1 = pltpu.async_copy(col4.at[c, s, pl.ds(gg0, glen)],
                                  colb.at[pl.ds(0, glen)], gs[0])
            d2 = pltpu.async_copy(row3.at[s, pl.ds(gg0, glen)],
                                  rowb.at[pl.ds(0, glen)], gs[1])
            d3 = pltpu.async_copy(w3.at[s, pl.ds(gg0, glen)],
                                  wb.at[pl.ds(0, glen)], gs[2])
            d1.wait(); d2.wait(); d3.wait()

        def pipe_group(gg0, glen):
            load_group(gg0, glen)
            pltpu.async_copy(src.at[colb.at[0]], m[0], gs[0])
            pltpu.async_copy(src.at[colb.at[1]], m[1], gs[1])

            def quad(i, _):
                for t in range(NBUF):
                    j = i * NBUF + t
                    n = (t + 2) % NBUF
                    pltpu.make_async_copy(src.at[colb.at[j]], m[t], gs[t]).wait()

                    if t < 2:
                        @pl.when(j >= 2)
                        def _():
                            pltpu.make_async_copy(
                                m[n], acc.at[rowb.at[j - 2]], ss[n]).wait()
                    else:
                        pltpu.make_async_copy(
                            m[n], acc.at[rowb.at[j - 2]], ss[n]).wait()

                    @pl.when(j + 2 < glen)
                    def _():
                        pltpu.async_copy(src.at[colb.at[j + 2]], m[n], gs[n])

                    scale_chunk(m[t], j)
                    pltpu.async_copy(m[t], acc.at[rowb.at[j]], ss[t], add=True)
                return _

            lax.fori_loop(0, glen // NBUF, quad, None)
            pltpu.make_async_copy(m[2], acc.at[rowb.at[glen - 2]], ss[2]).wait()
            pltpu.make_async_copy(m[3], acc.at[rowb.at[glen - 1]], ss[3]).wait()

        def grp_body(grp, _):
            pipe_group(pl.multiple_of(grp * GRP, GRP), GRP)
            return _

        lax.fori_loop(0, N_GRP, grp_body, None)
        pipe_group(N_GRP * GRP, TAILG)
        plsc.subcore_barrier()

        lax.fori_loop(0, RCH, zfill, None)

        if layer < N_LAYERS - 1:
            dst = (x1b, x2b)[layer]

            my_chunks(N_RCH, RPASS, RCH,
                      lambda r0: pltpu.async_copy(
                          acc.at[pl.ds(r0, RCH)],
                          dst.at[pl.ds(base + r0, RCH)], s0s))

            def drain_and_zero(r0):
                pltpu.make_async_copy(acc.at[pl.ds(r0, RCH)],
                                      dst.at[pl.ds(base + r0, RCH)], s0s).wait()
                pltpu.async_copy(m3.at[pl.ds(0, RCH)],
                                 acc.at[pl.ds(r0, RCH)], s1s)

            my_chunks(N_RCH, RPASS, RCH, drain_and_zero)
            my_chunks(N_RCH, RPASS, RCH,
                      lambda r0: pltpu.make_async_copy(
                          m3.at[pl.ds(0, RCH)],
                          acc.at[pl.ds(r0, RCH)], s1s).wait())
            plsc.subcore_barrier()

    quarter = jnp.full((16,), 0.25, jnp.float32)

    def mean_chunk(r0):
        d0 = pltpu.async_copy(x0.at[pl.ds(base + r0, RCH)], m0.at[pl.ds(0, RCH)], g0s)
        d1 = pltpu.async_copy(x1b.at[pl.ds(base + r0, RCH)], m1.at[pl.ds(0, RCH)], g1s)
        d2 = pltpu.async_copy(x2b.at[pl.ds(base + r0, RCH)], m2.at[pl.ds(0, RCH)], g2s)
        d3 = pltpu.async_copy(acc.at[pl.ds(r0, RCH)], m3.at[pl.ds(0, RCH)], g3s)
        d0.wait(); d1.wait(); d2.wait(); d3.wait()

        def mean_row(i, _):
            for g in range(HALF // 16):
                sl = pl.ds(g * 16, 16)
                m0[i, sl] = (m0[i, sl] + m1[i, sl] + m2[i, sl] + m3[i, sl]) * quarter
            return _

        lax.fori_loop(0, RCH, mean_row, None)
        pltpu.sync_copy(m0.at[pl.ds(0, RCH)], out.at[pl.ds(base + r0, RCH)])

    my_chunks(N_RCH, RPASS, RCH, mean_chunk)


_mesh = plsc.VectorSubcoreMesh(core_axis_name="c", subcore_axis_name="s")

_gbsr = pl.kernel(
    _body,
    out_type=[
        jax.ShapeDtypeStruct((2 * N_NODES, HALF), jnp.float32),
        jax.ShapeDtypeStruct((2 * N_NODES, HALF), jnp.float32),
        jax.ShapeDtypeStruct((2 * N_NODES, HALF), jnp.float32),
    ],
    mesh=_mesh,
    scratch_types=[
        pltpu.VMEM_SHARED((N_NODES, HALF), jnp.float32),
        pltpu.VMEM((CHUNK, HALF), jnp.float32),
        pltpu.VMEM((CHUNK, HALF), jnp.float32),
        pltpu.VMEM((CHUNK, HALF), jnp.float32),
        pltpu.VMEM((CHUNK, HALF), jnp.float32),
        pltpu.VMEM((GRP, CHUNK), jnp.int32),
        pltpu.VMEM((GRP, CHUNK), jnp.int32),
        pltpu.VMEM((GRP, CHUNK), jnp.float32),
        pltpu.SemaphoreType.DMA,
        pltpu.SemaphoreType.DMA,
        pltpu.SemaphoreType.DMA,
        pltpu.SemaphoreType.DMA,
        pltpu.SemaphoreType.DMA,
        pltpu.SemaphoreType.DMA,
        pltpu.SemaphoreType.DMA,
        pltpu.SemaphoreType.DMA,
    ],
)


def kernel(edge_index, edge_weight, user_emb, item_emb):
    ego = jnp.concatenate([user_emb, item_emb], axis=0)
    x0 = ego.reshape(N_NODES, 2, HALF).transpose(1, 0, 2).reshape(2 * N_NODES, HALF)
    pad = ((0, 0), (0, N_CHUNKS - N_REAL), (0, 0))
    col3 = jnp.pad(edge_index[1].astype(jnp.int32)
                   .reshape(TILES, N_REAL, CHUNK), pad)
    col4 = jnp.stack([col3, col3 + N_NODES])
    row3 = jnp.pad(edge_index[0].astype(jnp.int32)
                   .reshape(TILES, N_REAL, CHUNK), pad)
    w3 = jnp.pad(edge_weight.astype(jnp.float32)
                 .reshape(TILES, N_REAL, CHUNK), pad)
    out_dm, _x1, _x2 = _gbsr(x0, col4, row3, w3)
    mean = out_dm.reshape(2, N_NODES, HALF).transpose(1, 0, 2).reshape(N_NODES, LATENT_DIM)
    return (mean[:NUM_USER], mean[NUM_USER:])


# speedup vs baseline: 1.9018x; 1.9018x over previous
import jax
import jax.numpy as jnp
from jax import lax
from jax.experimental import pallas as pl
from jax.experimental.pallas import tpu as pltpu
from jax.experimental.pallas import tpu_sc as plsc

NUM_USER = 6000
NUM_ITEM = 4000
N_NODES = NUM_USER + NUM_ITEM
LATENT_DIM = 256
HALF = LATENT_DIM // 2
N_EDGES = 160000
TILES = 16
EDGES_PER_TILE = N_EDGES // TILES
CHUNK = 80
N_REAL = EDGES_PER_TILE // CHUNK
N_CHUNKS = 128
GRP = 24
N_GRP = 5
TAILG = N_CHUNKS - N_GRP * GRP
NBUF = 4
RCH = 40
N_RCH = N_NODES // RCH
RPASS = (N_RCH + TILES - 1) // TILES
N_LAYERS = 3


def _body(x0, col4, row3, w3, out, x1b, x2b,
          acc, m0, m1, m2, m3, colb, rowb, wb,
          g0s, g1s, g2s, g3s, s0s, s1s, s2s, s3s):
    c = lax.axis_index("c")
    s = lax.axis_index("s")
    base = c * N_NODES

    m = (m0, m1, m2, m3)
    gs = (g0s, g1s, g2s, g3s)
    ss = (s0s, s1s, s2s, s3s)

    zero16 = jnp.zeros((16,), jnp.float32)

    def zfill(i, _):
        for g in range(HALF // 16):
            m3[i, pl.ds(g * 16, 16)] = zero16
        return _

    def my_chunks(n_ch, npass, step, fn):
        for k in range(npass):
            ch = k * TILES + s
            @pl.when(ch < n_ch)
            def _():
                fn(pl.multiple_of(ch * step, step))

    lax.fori_loop(0, RCH, zfill, None)
    my_chunks(N_RCH, RPASS, RCH,
              lambda r0: pltpu.async_copy(m3.at[pl.ds(0, RCH)],
                                          acc.at[pl.ds(r0, RCH)], s0s))
    my_chunks(N_RCH, RPASS, RCH,
              lambda r0: pltpu.make_async_copy(m3.at[pl.ds(0, RCH)],
                                               acc.at[pl.ds(r0, RCH)], s0s).wait())
    plsc.subcore_barrier()

    def scale_chunk(buf, j):
        def scale16(q, _):
            w16 = wb[j, pl.ds(q * 16, 16)]
            for t in range(16):
                wbv = jnp.full((16,), w16[t], jnp.float32)
                e = q * 16 + t
                for g in range(HALF // 16):
                    sl = pl.ds(g * 16, 16)
                    buf[e, sl] = buf[e, sl] * wbv
            return _

        lax.fori_loop(0, CHUNK // 16, scale16, None)

    for layer in range(N_LAYERS):
        src = (x0, x1b, x2b)[layer]

        def load_group(gg0, glen):
            d1 = pltpu.async_copy(col4.at[c, s, pl.ds(gg0, glen)],
                                  colb.at[pl.ds(0, glen)], gs[0])
            d2 = pltpu.async_copy(row3.at[s, pl.ds(gg0, glen)],
                                  rowb.at[pl.ds(0, glen)], gs[1])
            d3 = pltpu.async_copy(w3.at[s, pl.ds(gg0, glen)],
                                  wb.at[pl.ds(0, glen)], gs[2])
            d1.wait(); d2.wait(); d3.wait()

        def pipe_group(gg0, glen):
            load_group(gg0, glen)
            pltpu.async_copy(src.at[colb.at[0]], m[0], gs[0])
            pltpu.async_copy(src.at[colb.at[1]], m[1], gs[1])

            def quad(i, _):
                for t in range(NBUF):
                    j = i * NBUF + t
                    n = (t + 2) % NBUF
                    pltpu.make_async_copy(src.at[colb.at[j]], m[t], gs[t]).wait()

                    if t < 2:
                        @pl.when(j >= 2)
                        def _():
                            pltpu.make_async_copy(
                                m[n], acc.at[rowb.at[j - 2]], ss[n]).wait()
                    else:
                        pltpu.make_async_copy(
                            m[n], acc.at[rowb.at[j - 2]], ss[n]).wait()

                    @pl.when(j + 2 < glen)
                    def _():
                        pltpu.async_copy(src.at[colb.at[j + 2]], m[n], gs[n])

                    scale_chunk(m[t], j)
                    pltpu.async_copy(m[t], acc.at[rowb.at[j]], ss[t], add=True)
                return _

            lax.fori_loop(0, glen // NBUF, quad, None)
            pltpu.make_async_copy(m[2], acc.at[rowb.at[glen - 2]], ss[2]).wait()
            pltpu.make_async_copy(m[3], acc.at[rowb.at[glen - 1]], ss[3]).wait()

        def grp_body(grp, _):
            pipe_group(pl.multiple_of(grp * GRP, GRP), GRP)
            return _

        lax.fori_loop(0, N_GRP, grp_body, None)
        pipe_group(N_GRP * GRP, TAILG)
        plsc.subcore_barrier()

        lax.fori_loop(0, RCH, zfill, None)

        if layer < N_LAYERS - 1:
            dst = (x1b, x2b)[layer]

            my_chunks(N_RCH, RPASS, RCH,
                      lambda r0: pltpu.async_copy(
                          acc.at[pl.ds(r0, RCH)],
                          dst.at[pl.ds(base + r0, RCH)], s0s))

            def drain_and_zero(r0):
                pltpu.make_async_copy(acc.at[pl.ds(r0, RCH)],
                                      dst.at[pl.ds(base + r0, RCH)], s0s).wait()
                pltpu.async_copy(m3.at[pl.ds(0, RCH)],
                                 acc.at[pl.ds(r0, RCH)], s1s)

            my_chunks(N_RCH, RPASS, RCH, drain_and_zero)
            my_chunks(N_RCH, RPASS, RCH,
                      lambda r0: pltpu.make_async_copy(
                          m3.at[pl.ds(0, RCH)],
                          acc.at[pl.ds(r0, RCH)], s1s).wait())
            plsc.subcore_barrier()

    quarter = jnp.full((16,), 0.25, jnp.float32)

    def mean_chunk(r0):
        d0 = pltpu.async_copy(x0.at[pl.ds(base + r0, RCH)], m0.at[pl.ds(0, RCH)], g0s)
        d1 = pltpu.async_copy(x1b.at[pl.ds(base + r0, RCH)], m1.at[pl.ds(0, RCH)], g1s)
        d2 = pltpu.async_copy(x2b.at[pl.ds(base + r0, RCH)], m2.at[pl.ds(0, RCH)], g2s)
        d3 = pltpu.async_copy(acc.at[pl.ds(r0, RCH)], m3.at[pl.ds(0, RCH)], g3s)
        d0.wait(); d1.wait(); d2.wait(); d3.wait()

        def mean_row(i, _):
            for g in range(HALF // 16):
                sl = pl.ds(g * 16, 16)
                m0[i, sl] = (m0[i, sl] + m1[i, sl] + m2[i, sl] + m3[i, sl]) * quarter
            return _

        lax.fori_loop(0, RCH, mean_row, None)
        pltpu.sync_copy(m0.at[pl.ds(0, RCH)], out.at[pl.ds(base + r0, RCH)])

    my_chunks(N_RCH, RPASS, RCH, mean_chunk)


_mesh = plsc.VectorSubcoreMesh(core_axis_name="c", subcore_axis_name="s")

_gbsr = pl.kernel(
    _body,
    out_type=[
        jax.ShapeDtypeStruct((2 * N_NODES, HALF), jnp.float32),
        jax.ShapeDtypeStruct((2 * N_NODES, HALF), jnp.float32),
        jax.ShapeDtypeStruct((2 * N_NODES, HALF), jnp.float32),
    ],
    mesh=_mesh,
    scratch_types=[
        pltpu.VMEM_SHARED((N_NODES, HALF), jnp.float32),
        pltpu.VMEM((CHUNK, HALF), jnp.float32),
        pltpu.VMEM((CHUNK, HALF), jnp.float32),
        pltpu.VMEM((CHUNK, HALF), jnp.float32),
        pltpu.VMEM((CHUNK, HALF), jnp.float32),
        pltpu.VMEM((GRP, CHUNK), jnp.int32),
        pltpu.VMEM((GRP, CHUNK), jnp.int32),
        pltpu.VMEM((GRP, CHUNK), jnp.float32),
        pltpu.SemaphoreType.DMA,
        pltpu.SemaphoreType.DMA,
        pltpu.SemaphoreType.DMA,
        pltpu.SemaphoreType.DMA,
        pltpu.SemaphoreType.DMA,
        pltpu.SemaphoreType.DMA,
        pltpu.SemaphoreType.DMA,
        pltpu.SemaphoreType.DMA,
    ],
)


def kernel(edge_index, edge_weight, user_emb, item_emb):
    ego = jnp.concatenate([user_emb, item_emb], axis=0)
    x0 = ego.reshape(N_NODES, 2, HALF).transpose(1, 0, 2).reshape(2 * N_NODES, HALF)
    npad = N_CHUNKS - N_REAL
    col3 = edge_index[1].astype(jnp.int32).reshape(TILES, N_REAL, CHUNK)
    col3 = jnp.concatenate([col3, col3[:, :npad]], axis=1)
    col4 = jnp.stack([col3, col3 + N_NODES])
    row3 = edge_index[0].astype(jnp.int32).reshape(TILES, N_REAL, CHUNK)
    row3 = jnp.concatenate([row3, row3[:, :npad]], axis=1)
    w3 = edge_weight.astype(jnp.float32).reshape(TILES, N_REAL, CHUNK)
    w3 = jnp.concatenate([w3, jnp.zeros((TILES, npad, CHUNK), jnp.float32)],
                         axis=1)
    out_dm, _x1, _x2 = _gbsr(x0, col4, row3, w3)
    mean = out_dm.reshape(2, N_NODES, HALF).transpose(1, 0, 2).reshape(N_NODES, LATENT_DIM)
    return (mean[:NUM_USER], mean[NUM_USER:])
